# baseline (speedup 1.0000x reference)
import functools

import jax
import jax.numpy as jnp
from jax.experimental import pallas as pl
from jax.experimental.pallas import tpu as pltpu


def _round_up(n, m):
    return ((n + m - 1) // m) * m


def _cdiv(a, b):
    return (a + b - 1) // b


def _mlp_chunked_kernel(chunks, x_ref, w1_ref, b1_ref, w2_ref, b2_ref,
                        w3_ref, b3_ref, o_ref):
    w1 = w1_ref[...]
    w2 = w2_ref[...]
    w3 = w3_ref[...]
    b1 = b1_ref[...]
    b2 = b2_ref[...]
    b3 = b3_ref[...]
    base = 0
    for cm in chunks:
        rows = pl.ds(base, cm)
        base += cm
        xc = x_ref[rows, :]
        h1 = jnp.dot(xc, w1, preferred_element_type=jnp.float32) + b1
        h1 = jnp.maximum(h1, 0.0)
        h2 = jnp.dot(h1, w2, preferred_element_type=jnp.float32) + b2
        h2 = jnp.maximum(h2, 0.0)
        out = jnp.dot(h2, w3, preferred_element_type=jnp.float32) + b3
        o_ref[rows, :] = out.astype(o_ref.dtype)


def kernel(x, W1, b1, W2, b2, W3, b3):
    B, state_size = x.shape
    d_out = W3.shape[1]

    tile_b = min(4096, _round_up(_cdiv(B, 2), 8))
    b_pad = _round_up(B, tile_b)
    x_in = x if b_pad == B else jnp.pad(x, ((0, b_pad - B), (0, 0)))

    if tile_b % 512 == 0 and tile_b >= 1024:
        chunks = [512] * (tile_b // 512 - 1) + [256, 128, 128]
    else:
        chunks = [tile_b]

    weights = (W1, b1, W2, b2, W3, b3)
    act_spec = pl.BlockSpec((tile_b, state_size), lambda i: (i, 0))
    out_spec = pl.BlockSpec((tile_b, d_out), lambda i: (i, 0))

    def resident(a):
        return pl.BlockSpec(a.shape, lambda i: (0,) * a.ndim)

    out_pad = pl.pallas_call(
        functools.partial(_mlp_chunked_kernel, tuple(chunks)),
        out_shape=jax.ShapeDtypeStruct((b_pad, d_out), jnp.float32),
        grid=(b_pad // tile_b,),
        in_specs=[act_spec] + [resident(w) for w in weights],
        out_specs=out_spec,
        compiler_params=pltpu.CompilerParams(
            dimension_semantics=("parallel",)),
    )(x_in, *weights)

    return out_pad[:B, :d_out]


# speedup vs baseline: 1.0209x; 1.0209x over previous
import functools

import jax
import jax.numpy as jnp
from jax.experimental import pallas as pl
from jax.experimental.pallas import tpu as pltpu


def _round_up(n, m):
    return ((n + m - 1) // m) * m


def _cdiv(a, b):
    return (a + b - 1) // b


def _mlp_chunked_kernel(chunks, x_ref, w1_ref, b1_ref, w2_ref, b2_ref,
                        w3_ref, b3_ref, o_ref):
    w1 = w1_ref[...]
    w2 = w2_ref[...]
    w3 = w3_ref[...]
    b1 = b1_ref[...]
    b2 = b2_ref[...]
    b3 = b3_ref[...]
    base = 0
    for cm in chunks:
        rows = pl.ds(base, cm)
        base += cm
        xc = x_ref[rows, :]
        h1 = jnp.dot(xc, w1, preferred_element_type=jnp.float32) + b1
        h1 = jnp.maximum(h1, 0.0)
        h2 = jnp.dot(h1, w2, preferred_element_type=jnp.float32) + b2
        h2 = jnp.maximum(h2, 0.0)
        out = jnp.dot(h2, w3, preferred_element_type=jnp.float32) + b3
        o_ref[rows, :] = out.astype(o_ref.dtype)


def kernel(x, W1, b1, W2, b2, W3, b3):
    B, state_size = x.shape
    d_out = W3.shape[1]

    tile_b = min(4096, _round_up(_cdiv(B, 2), 8))
    b_pad = _round_up(B, tile_b)
    x_in = x if b_pad == B else jnp.pad(x, ((0, b_pad - B), (0, 0)))

    if tile_b % 512 == 0 and tile_b >= 1024:
        chunks = [512] * (tile_b // 512 - 1) + [256, 256]
    else:
        chunks = [tile_b]

    weights = (W1, b1, W2, b2, W3, b3)
    act_spec = pl.BlockSpec((tile_b, state_size), lambda i: (i, 0))
    out_spec = pl.BlockSpec((tile_b, d_out), lambda i: (i, 0))

    def resident(a):
        return pl.BlockSpec(a.shape, lambda i: (0,) * a.ndim)

    out_pad = pl.pallas_call(
        functools.partial(_mlp_chunked_kernel, tuple(chunks)),
        out_shape=jax.ShapeDtypeStruct((b_pad, d_out), jnp.float32),
        grid=(b_pad // tile_b,),
        in_specs=[act_spec] + [resident(w) for w in weights],
        out_specs=out_spec,
        compiler_params=pltpu.CompilerParams(
            dimension_semantics=("parallel",)),
    )(x_in, *weights)

    return out_pad[:B, :d_out]
